# baseline (speedup 1.0000x reference)
import functools

import jax
import jax.numpy as jnp
from jax import lax
from jax.experimental import pallas as pl
from jax.experimental.pallas import tpu as pltpu
from jax.experimental.pallas import tpu_sc as plsc

_WS = 32
_D = 2 * _WS - 1
_H = 32
_N = _WS * _WS


def _sc_body(tp_hbm, out_hbm, tbl_v, tall_v, sem):
    h = lax.axis_index("s") * 2 + lax.axis_index("c")
    pltpu.sync_copy(tp_hbm.at[h], tbl_v)

    def build_j1(j1, _):
        w = _WS - 1 - j1

        def build_e(e, _):
            src_row = 2 * _WS - 1 - e
            base = j1 * 2 * _N + e * _WS
            tall_v[pl.ds(base, 16)] = tbl_v[src_row, pl.ds(w, 16)]
            tall_v[pl.ds(base + 16, 16)] = tbl_v[src_row, pl.ds(w + 16, 16)]
            return 0

        return lax.fori_loop(0, 64, build_e, 0)

    lax.fori_loop(0, _WS, build_j1, 0)

    def fire(k, _):
        i1 = k // _WS
        j1 = k % _WS
        pltpu.async_copy(
            tall_v.at[pl.ds(j1 * 2 * _N + (_WS - i1) * _WS, _N)],
            out_hbm.at[pl.ds(((h * _WS + i1) * _WS + j1) * _N, _N)],
            sem,
        )
        return 0

    lax.fori_loop(0, _N, fire, 0)

    def drain(k, _):
        i1 = k // _WS
        j1 = k % _WS
        pltpu.make_async_copy(
            tall_v.at[pl.ds(j1 * 2 * _N + (_WS - i1) * _WS, _N)],
            out_hbm.at[pl.ds(((h * _WS + i1) * _WS + j1) * _N, _N)],
            sem,
        ).wait()
        return 0

    lax.fori_loop(0, _N, drain, 0)


def kernel(bias_table, relative_position_index):
    del relative_position_index
    t3 = bias_table.reshape(_D, _D, _H)
    tp = jnp.flip(t3, axis=1).transpose(2, 0, 1)
    tp = jnp.pad(tp, ((0, 0), (0, 64 - _D), (0, 128 - _D)))

    mesh = plsc.VectorSubcoreMesh(core_axis_name="c", subcore_axis_name="s")
    sck = functools.partial(
        pl.kernel,
        mesh=mesh,
        out_type=jax.ShapeDtypeStruct((_H * _N * _N,), jnp.float32),
        scratch_types=[
            pltpu.VMEM((64, 128), jnp.float32),
            pltpu.VMEM((_WS * 2 * _N,), jnp.float32),
            pltpu.SemaphoreType.DMA,
        ],
    )(_sc_body)
    out_flat = sck(tp)
    return out_flat.reshape(_H, _N, _N)


# speedup vs baseline: 1.0661x; 1.0661x over previous
import functools

import jax
import jax.numpy as jnp
from jax import lax
from jax.experimental import pallas as pl
from jax.experimental.pallas import tpu as pltpu
from jax.experimental.pallas import tpu_sc as plsc

_WS = 32
_D = 2 * _WS - 1
_H = 32
_N = _WS * _WS


def _sc_body(tp_hbm, out_hbm, tbl_v, tall_v, sem):
    h = lax.axis_index("s") * 2 + lax.axis_index("c")
    pltpu.sync_copy(tp_hbm.at[h], tbl_v)

    out_base = h * _N * _N

    def per_j1(j1, _):
        w = _WS - 1 - j1
        base_j = j1 * 2 * _N

        def build_e(e, _):
            src_row = 2 * _WS - 1 - e
            base = base_j + e * _WS
            tall_v[pl.ds(base, 16)] = tbl_v[src_row, pl.ds(w, 16)]
            tall_v[pl.ds(base + 16, 16)] = tbl_v[src_row, pl.ds(w + 16, 16)]
            return 0

        lax.fori_loop(0, 64, build_e, 0, unroll=8)

        def fire_i1(i1, _):
            pltpu.async_copy(
                tall_v.at[pl.ds(base_j + (_WS - i1) * _WS, _N)],
                out_hbm.at[pl.ds(out_base + i1 * _WS * _N + j1 * _N, _N)],
                sem,
            )
            return 0

        return lax.fori_loop(0, _WS, fire_i1, 0, unroll=4)

    lax.fori_loop(0, _WS, per_j1, 0)

    def drain(k, _):
        pltpu.make_async_copy(
            tall_v.at[pl.ds(0, _N)],
            out_hbm.at[pl.ds(out_base + k * _N, _N)],
            sem,
        ).wait()
        return 0

    lax.fori_loop(0, _N, drain, 0, unroll=4)


def kernel(bias_table, relative_position_index):
    del relative_position_index
    t3 = bias_table.reshape(_D, _D, _H)
    tp = jnp.flip(t3, axis=1).transpose(2, 0, 1)
    tp = jnp.pad(tp, ((0, 0), (0, 64 - _D), (0, 128 - _D)))

    mesh = plsc.VectorSubcoreMesh(core_axis_name="c", subcore_axis_name="s")
    sck = functools.partial(
        pl.kernel,
        mesh=mesh,
        out_type=jax.ShapeDtypeStruct((_H * _N * _N,), jnp.float32),
        scratch_types=[
            pltpu.VMEM((64, 128), jnp.float32),
            pltpu.VMEM((_WS * 2 * _N,), jnp.float32),
            pltpu.SemaphoreType.DMA,
        ],
    )(_sc_body)
    out_flat = sck(tp)
    return out_flat.reshape(_H, _N, _N)
